# baseline (speedup 1.0000x reference)
import functools

import jax
import jax.numpy as jnp
from jax import lax
from jax.experimental import pallas as pl
from jax.experimental.pallas import tpu as pltpu
from jax.experimental.pallas import tpu_sc as plsc

B, N, C, D, K, NCLS = 4, 2048, 128, 128, 16, 40
BN = B * N
BNK = B * N * K
XP = 16

def _k0_body(wq, wk, wg1, wp2, bp2, bg1, wqg, wkg, wp2cat, bcat):
    g1 = wg1[...]
    wqg[...] = jnp.dot(wq[...], g1, preferred_element_type=jnp.float32)
    wkg[...] = jnp.dot(wk[...], g1, preferred_element_type=jnp.float32)
    p2 = wp2[...]
    p2g = jnp.dot(p2, g1, preferred_element_type=jnp.float32)
    wp2cat[...] = jnp.concatenate([p2, p2g], axis=1)
    b2 = bp2[...]
    b2g = jnp.dot(b2, g1, preferred_element_type=jnp.float32) + bg1[...]
    bcat[...] = jnp.concatenate([b2, b2g], axis=1)


def _combine_weights(Wq, Wk, Wg1, Wp2, bp2, bg1):
    f32 = jnp.float32
    return pl.pallas_call(
        _k0_body,
        out_shape=(
            jax.ShapeDtypeStruct((D, D), f32),
            jax.ShapeDtypeStruct((D, D), f32),
            jax.ShapeDtypeStruct((D, 2 * D), f32),
            jax.ShapeDtypeStruct((1, 2 * D), f32),
        ),
    )(Wq, Wk, Wg1, Wp2, bp2.reshape(1, D), bg1.reshape(1, D))


PB1 = 512

def _k1_body(f_ref, xyzp_ref, we, be, wqg, wp1, x_o, qg_o, p_o):
    x = jnp.dot(f_ref[...], we[...], preferred_element_type=jnp.float32) + be[...]
    x_o[...] = x
    qg_o[...] = jnp.dot(x, wqg[...], preferred_element_type=jnp.float32)
    p_o[...] = jnp.dot(xyzp_ref[...], wp1[...],
                       preferred_element_type=jnp.float32)


def _project(f_flat, xyzp_flat, W_embed, b_embed, Wqg, Wp1p):
    f32 = jnp.float32
    blk = pl.BlockSpec((PB1, D), lambda i: (i, 0))
    wspec = pl.BlockSpec((D, D), lambda i: (0, 0))
    return pl.pallas_call(
        _k1_body,
        grid=(BN // PB1,),
        in_specs=[blk, pl.BlockSpec((PB1, XP), lambda i: (i, 0)),
                  wspec, pl.BlockSpec((1, D), lambda i: (0, 0)),
                  wspec, pl.BlockSpec((XP, D), lambda i: (0, 0))],
        out_specs=(blk, blk, blk),
        out_shape=tuple(jax.ShapeDtypeStruct((BN, D), f32) for _ in range(3)),
    )(f_flat, xyzp_flat, W_embed, b_embed.reshape(1, D), Wqg, Wp1p)


PB2 = 256

def _k2_body(xyz_blk_ref, xyzt_ref, sq_ref, idx_o, *, base_b):
    b = pl.program_id(0) + base_b
    xb = xyz_blk_ref[0]
    sq_all = sq_ref[0]
    sq_blk = jnp.sum(xb * xb, axis=1, keepdims=True)
    cross = jnp.dot(xb, xyzt_ref[0], preferred_element_type=jnp.float32)
    d2 = jnp.maximum(sq_blk + sq_all - 2.0 * cross, 0.0)
    col = lax.broadcasted_iota(jnp.int32, (PB2, N), 1)
    key = (lax.bitcast_convert_type(d2, jnp.int32) & jnp.int32(~2047)) | col
    big = jnp.int32(2147483647)

    NC = N // 16
    slices = [key[:, s * NC:(s + 1) * NC] for s in range(16)]
    m1 = slices[0]
    for s in range(1, 16):
        m1 = jnp.minimum(m1, slices[s])
    m2 = m3 = m4 = None
    prev = slices
    for lvl in range(3):
        cur_min = (m1, m2, m3)[lvl]
        nxt = [jnp.where(sl == cur_min, big, sl) for sl in prev]
        mm = nxt[0]
        for s in range(1, 16):
            mm = jnp.minimum(mm, nxt[s])
        if lvl == 0:
            m2 = mm
        elif lvl == 1:
            m3 = mm
        else:
            m4 = mm
        prev = nxt
    pool = jnp.concatenate([m1, m2, m3, m4], axis=1)
    ams = []
    for _ in range(K):
        m = jnp.min(pool, axis=1, keepdims=True)
        ams.append(m)
        pool = jnp.where(pool == m, big, pool)
    fast = jnp.concatenate(ams, axis=1)
    bad = jnp.any(m4 < ams[-1])

    def _slow(kk):
        outs = []
        for _ in range(K):
            mm = jnp.min(kk, axis=1, keepdims=True)
            outs.append(mm)
            kk = jnp.where(kk == mm, big, kk)
        return jnp.concatenate(outs, axis=1)

    keys16 = lax.cond(bad, lambda: _slow(key), lambda: fast)
    idx_o[0] = (keys16 & jnp.int32(2047)) + b * N


def _knn(xyz_pad, base_b):
    Bh = xyz_pad.shape[0]
    xyzt = jnp.swapaxes(xyz_pad, 1, 2)
    sq = jnp.sum(xyz_pad * xyz_pad, axis=2)[:, None, :]
    return pl.pallas_call(
        functools.partial(_k2_body, base_b=base_b),
        grid=(Bh, N // PB2),
        in_specs=[
            pl.BlockSpec((1, PB2, XP), lambda b, i: (b, i, 0)),
            pl.BlockSpec((1, XP, N), lambda b, i: (b, 0, 0)),
            pl.BlockSpec((1, 1, N), lambda b, i: (b, 0, 0)),
        ],
        out_specs=pl.BlockSpec((1, PB2, K), lambda b, i: (b, i, 0)),
        out_shape=jax.ShapeDtypeStruct((Bh, N, K), jnp.int32),
    )(xyz_pad, xyzt, sq)


GW = 128

def _sc_gather(x, p, gidx):
    f32 = jnp.float32
    nidx = gidx.shape[0]
    mesh = plsc.VectorSubcoreMesh(core_axis_name="c", subcore_axis_name="s")
    idx2 = gidx.reshape(1, nidx)

    @functools.partial(
        pl.kernel,
        out_type=(
            jax.ShapeDtypeStruct((nidx, D), f32),
            jax.ShapeDtypeStruct((nidx, D), f32),
        ),
        mesh=mesh,
    )
    def gather_kernel(x_hbm, p_hbm, i_hbm, ox_hbm, op_hbm):
        def body(i_vmem, ox_vmem, op_vmem):
            pltpu.sync_copy(x_hbm.at[i_vmem.at[0]], ox_vmem)
            pltpu.sync_copy(p_hbm.at[i_vmem.at[0]], op_vmem)

        pltpu.emit_pipeline(
            body,
            grid=(nidx // GW,),
            in_specs=[pl.BlockSpec((1, GW), lambda i: (0, i))],
            out_specs=[
                pl.BlockSpec((GW, D), lambda i: (i, 0)),
                pl.BlockSpec((GW, D), lambda i: (i, 0)),
            ],
            core_axis_name=("c", "s"),
            dimension_semantics=(pltpu.PARALLEL,),
        )(i_hbm, ox_hbm, op_hbm)

    return gather_kernel(x, p, idx2)


PB4 = 128

def _k4_body(x_ref, qg_ref, pb_ref, xr_ref, pr_ref,
             wkg, wv, bp1, wp2cat, bcat, wg2, bg2, pm_ref):
    f32 = jnp.float32
    bf16 = jnp.bfloat16
    xr = xr_ref[...].astype(bf16)
    kgr = jnp.dot(xr, wkg[...], preferred_element_type=f32)
    vr = jnp.dot(xr, wv[...], preferred_element_type=f32)
    p_i = pb_ref[...].reshape(PB4, 1, D) + bp1[...]
    h1 = jnp.maximum(
        (p_i - pr_ref[...].reshape(PB4, K, D)).reshape(PB4 * K, D), 0.0)
    hcat = jnp.dot(h1.astype(bf16), wp2cat[...],
                   preferred_element_type=f32) + bcat[...]
    pe = hcat[:, :D]
    pg = hcat[:, D:]
    qgb = jnp.broadcast_to(qg_ref[...].reshape(PB4, 1, D),
                           (PB4, K, D)).reshape(PB4 * K, D)
    a = jnp.maximum(qgb - kgr + pg, 0.0)
    u = jnp.dot(a.astype(bf16), wg2[...], preferred_element_type=f32) + bg2[...]
    u3 = u.reshape(PB4, K, D)
    mx = jnp.max(u3, axis=1, keepdims=True)
    e = jnp.exp(u3 - mx)
    s = jnp.sum(e, axis=1, keepdims=True)
    attn = e / s
    contrib = attn * (vr + pe).reshape(PB4, K, D)
    out = jnp.sum(contrib, axis=1) + x_ref[...]
    pm_ref[...] = jnp.max(out, axis=0, keepdims=True)[None]


def _pair_stage(x, qg, p, xr, pr, base_blk, Wkg, Wv, bp1, Wp2cat, bcat, Wg2,
                bg2):
    f32 = jnp.float32
    bf16 = jnp.bfloat16
    Wkg, Wv, Wp2cat, Wg2 = (w.astype(bf16) for w in (Wkg, Wv, Wp2cat, Wg2))
    nblk = xr.shape[0] // (PB4 * K)
    blkp = pl.BlockSpec((PB4, D), lambda i: (base_blk + i, 0))
    blkr = pl.BlockSpec((PB4 * K, D), lambda i: (i, 0))
    wspec = pl.BlockSpec((D, D), lambda i: (0, 0))
    bspec = pl.BlockSpec((1, D), lambda i: (0, 0))
    return pl.pallas_call(
        _k4_body,
        grid=(nblk,),
        in_specs=[
            blkp, blkp, blkp,
            blkr, blkr,
            wspec, wspec, bspec,
            pl.BlockSpec((D, 2 * D), lambda i: (0, 0)),
            pl.BlockSpec((1, 2 * D), lambda i: (0, 0)),
            wspec, bspec,
        ],
        out_specs=pl.BlockSpec((1, 1, D), lambda i: (i, 0, 0)),
        out_shape=jax.ShapeDtypeStruct((nblk, 1, D), f32),
    )(x, qg, p, xr, pr, Wkg, Wv, bp1, Wp2cat, bcat, Wg2, bg2)


def _k5_body(pm_ref, wc, bc, o_ref):
    feat = jnp.max(pm_ref[...], axis=1)
    o_ref[...] = jnp.dot(feat, wc[...], preferred_element_type=jnp.float32) + bc[...]


def _classify(pmax, Wc, bc):
    return pl.pallas_call(
        _k5_body,
        out_shape=jax.ShapeDtypeStruct((B, NCLS), jnp.float32),
    )(pmax, Wc, bc.reshape(1, NCLS))


def kernel(features, xyz, W_embed, b_embed, Wq, Wk, Wv, Wp1, bp1, Wp2, bp2,
           Wg1, bg1, Wg2, bg2, Wc, bc):
    f32 = jnp.float32
    Wqg, Wkg, Wp2cat, bcat = _combine_weights(Wq, Wk, Wg1, Wp2, bp2, bg1)

    xyz_pad = jnp.pad(xyz, ((0, 0), (0, 0), (0, XP - 3)))
    xyzp_flat = xyz_pad.reshape(BN, XP)
    Wp1p = jnp.pad(Wp1, ((0, XP - 3), (0, 0)))

    f_flat = features.reshape(BN, C)
    x, qg, p = _project(f_flat, xyzp_flat, W_embed, b_embed, Wqg, Wp1p)

    HB = 1
    pmaxes = []
    for h in range(B // HB):
        gidx_h = _knn(xyz_pad[h * HB:(h + 1) * HB], base_b=h * HB)
        xr_h, pr_h = _sc_gather(x, p, gidx_h.reshape(HB * N * K))
        pmax_h = _pair_stage(x, qg, p, xr_h, pr_h, h * (HB * N // PB4),
                             Wkg, Wv, bp1.reshape(1, D), Wp2cat, bcat, Wg2,
                             bg2.reshape(1, D))
        pmaxes.append(pmax_h)
    pmax = jnp.concatenate(pmaxes, axis=0)
    logits = _classify(pmax.reshape(B, BN // PB4 // B, D), Wc, bc)
    return logits


# speedup vs baseline: 1.0353x; 1.0353x over previous
import functools

import jax
import jax.numpy as jnp
from jax import lax
from jax.experimental import pallas as pl
from jax.experimental.pallas import tpu as pltpu
from jax.experimental.pallas import tpu_sc as plsc

B, N, C, D, K, NCLS = 4, 2048, 128, 128, 16, 40
BN = B * N
BNK = B * N * K
XP = 16

def _k0_body(wq, wk, wg1, wp2, bp2, bg1, wqg, wkg, wp2cat, bcat):
    g1 = wg1[...]
    wqg[...] = jnp.dot(wq[...], g1, preferred_element_type=jnp.float32)
    wkg[...] = jnp.dot(wk[...], g1, preferred_element_type=jnp.float32)
    p2 = wp2[...]
    p2g = jnp.dot(p2, g1, preferred_element_type=jnp.float32)
    wp2cat[...] = jnp.concatenate([p2, p2g], axis=1)
    b2 = bp2[...]
    b2g = jnp.dot(b2, g1, preferred_element_type=jnp.float32) + bg1[...]
    bcat[...] = jnp.concatenate([b2, b2g], axis=1)


def _combine_weights(Wq, Wk, Wg1, Wp2, bp2, bg1):
    f32 = jnp.float32
    return pl.pallas_call(
        _k0_body,
        out_shape=(
            jax.ShapeDtypeStruct((D, D), f32),
            jax.ShapeDtypeStruct((D, D), f32),
            jax.ShapeDtypeStruct((D, 2 * D), f32),
            jax.ShapeDtypeStruct((1, 2 * D), f32),
        ),
    )(Wq, Wk, Wg1, Wp2, bp2.reshape(1, D), bg1.reshape(1, D))


PB1 = 2048

def _k1_body(f_ref, xyzp_ref, we, be, wqg, wp1, x_o, qg_o, p_o):
    x = jnp.dot(f_ref[...], we[...], preferred_element_type=jnp.float32) + be[...]
    x_o[...] = x
    qg_o[...] = jnp.dot(x, wqg[...], preferred_element_type=jnp.float32)
    p_o[...] = jnp.dot(xyzp_ref[...], wp1[...],
                       preferred_element_type=jnp.float32)


def _project(f_flat, xyzp_flat, W_embed, b_embed, Wqg, Wp1p):
    f32 = jnp.float32
    blk = pl.BlockSpec((PB1, D), lambda i: (i, 0))
    wspec = pl.BlockSpec((D, D), lambda i: (0, 0))
    return pl.pallas_call(
        _k1_body,
        grid=(BN // PB1,),
        in_specs=[blk, pl.BlockSpec((PB1, XP), lambda i: (i, 0)),
                  wspec, pl.BlockSpec((1, D), lambda i: (0, 0)),
                  wspec, pl.BlockSpec((XP, D), lambda i: (0, 0))],
        out_specs=(blk, blk, blk),
        out_shape=tuple(jax.ShapeDtypeStruct((BN, D), f32) for _ in range(3)),
    )(f_flat, xyzp_flat, W_embed, b_embed.reshape(1, D), Wqg, Wp1p)


PB2 = 256

def _k2_body(xyz_blk_ref, xyzt_ref, sq_ref, idx_o, *, base_b):
    b = pl.program_id(0) + base_b
    xb = xyz_blk_ref[0]
    sq_all = sq_ref[0]
    sq_blk = jnp.sum(xb * xb, axis=1, keepdims=True)
    cross = jnp.dot(xb, xyzt_ref[0], preferred_element_type=jnp.float32)
    d2 = jnp.maximum(sq_blk + sq_all - 2.0 * cross, 0.0)
    col = lax.broadcasted_iota(jnp.int32, (PB2, N), 1)
    key = (lax.bitcast_convert_type(d2, jnp.int32) & jnp.int32(~2047)) | col
    big = jnp.int32(2147483647)

    NC = N // 16
    slices = [key[:, s * NC:(s + 1) * NC] for s in range(16)]
    m1 = slices[0]
    for s in range(1, 16):
        m1 = jnp.minimum(m1, slices[s])
    m2 = m3 = m4 = None
    prev = slices
    for lvl in range(3):
        cur_min = (m1, m2, m3)[lvl]
        nxt = [jnp.where(sl == cur_min, big, sl) for sl in prev]
        mm = nxt[0]
        for s in range(1, 16):
            mm = jnp.minimum(mm, nxt[s])
        if lvl == 0:
            m2 = mm
        elif lvl == 1:
            m3 = mm
        else:
            m4 = mm
        prev = nxt
    pool = jnp.concatenate([m1, m2, m3, m4], axis=1)
    ams = []
    for _ in range(K):
        m = jnp.min(pool, axis=1, keepdims=True)
        ams.append(m)
        pool = jnp.where(pool == m, big, pool)
    fast = jnp.concatenate(ams, axis=1)
    bad = jnp.any(m4 < ams[-1])

    def _slow(kk):
        outs = []
        for _ in range(K):
            mm = jnp.min(kk, axis=1, keepdims=True)
            outs.append(mm)
            kk = jnp.where(kk == mm, big, kk)
        return jnp.concatenate(outs, axis=1)

    keys16 = lax.cond(bad, lambda: _slow(key), lambda: fast)
    idx_o[0] = (keys16 & jnp.int32(2047)) + b * N


def _knn(xyz_pad, base_b):
    Bh = xyz_pad.shape[0]
    xyzt = jnp.swapaxes(xyz_pad, 1, 2)
    sq = jnp.sum(xyz_pad * xyz_pad, axis=2)[:, None, :]
    return pl.pallas_call(
        functools.partial(_k2_body, base_b=base_b),
        grid=(Bh, N // PB2),
        in_specs=[
            pl.BlockSpec((1, PB2, XP), lambda b, i: (b, i, 0)),
            pl.BlockSpec((1, XP, N), lambda b, i: (b, 0, 0)),
            pl.BlockSpec((1, 1, N), lambda b, i: (b, 0, 0)),
        ],
        out_specs=pl.BlockSpec((1, PB2, K), lambda b, i: (b, i, 0)),
        out_shape=jax.ShapeDtypeStruct((Bh, N, K), jnp.int32),
    )(xyz_pad, xyzt, sq)


GW = 128

def _sc_gather(x, p, gidx):
    f32 = jnp.float32
    nidx = gidx.shape[0]
    mesh = plsc.VectorSubcoreMesh(core_axis_name="c", subcore_axis_name="s")
    idx2 = gidx.reshape(1, nidx)

    @functools.partial(
        pl.kernel,
        out_type=(
            jax.ShapeDtypeStruct((nidx, D), f32),
            jax.ShapeDtypeStruct((nidx, D), f32),
        ),
        mesh=mesh,
    )
    def gather_kernel(x_hbm, p_hbm, i_hbm, ox_hbm, op_hbm):
        def body(i_vmem, ox_vmem, op_vmem):
            pltpu.sync_copy(x_hbm.at[i_vmem.at[0]], ox_vmem)
            pltpu.sync_copy(p_hbm.at[i_vmem.at[0]], op_vmem)

        pltpu.emit_pipeline(
            body,
            grid=(nidx // GW,),
            in_specs=[pl.BlockSpec((1, GW), lambda i: (0, i))],
            out_specs=[
                pl.BlockSpec((GW, D), lambda i: (i, 0)),
                pl.BlockSpec((GW, D), lambda i: (i, 0)),
            ],
            core_axis_name=("c", "s"),
            dimension_semantics=(pltpu.PARALLEL,),
        )(i_hbm, ox_hbm, op_hbm)

    return gather_kernel(x, p, idx2)


PB4 = 128

def _k4_body(x_ref, qg_ref, pb_ref, xr_ref, pr_ref,
             wkg, wv, bp1, wp2cat, bcat, wg2, bg2, pm_ref):
    f32 = jnp.float32
    xr = xr_ref[...]
    kgr = jnp.dot(xr, wkg[...], preferred_element_type=f32)
    vr = jnp.dot(xr, wv[...], preferred_element_type=f32)
    p_i = pb_ref[...].reshape(PB4, 1, D) + bp1[...]
    h1 = jnp.maximum(
        (p_i - pr_ref[...].reshape(PB4, K, D)).reshape(PB4 * K, D), 0.0)
    hcat = jnp.dot(h1, wp2cat[...], preferred_element_type=f32) + bcat[...]
    pe = hcat[:, :D]
    pg = hcat[:, D:]
    qgb = jnp.broadcast_to(qg_ref[...].reshape(PB4, 1, D),
                           (PB4, K, D)).reshape(PB4 * K, D)
    a = jnp.maximum(qgb - kgr + pg, 0.0)
    u = jnp.dot(a, wg2[...], preferred_element_type=f32) + bg2[...]
    u3 = u.reshape(PB4, K, D)
    mx = jnp.max(u3, axis=1, keepdims=True)
    e = jnp.exp(u3 - mx)
    s = jnp.sum(e, axis=1)
    wsum = jnp.sum(e * (vr + pe).reshape(PB4, K, D), axis=1)
    out = wsum / s + x_ref[...]
    pm_ref[...] = jnp.max(out, axis=0, keepdims=True)[None]


def _pair_stage(x, qg, p, xr, pr, base_blk, Wkg, Wv, bp1, Wp2cat, bcat, Wg2,
                bg2):
    f32 = jnp.float32
    nblk = xr.shape[0] // (PB4 * K)
    blkp = pl.BlockSpec((PB4, D), lambda i: (base_blk + i, 0))
    blkr = pl.BlockSpec((PB4 * K, D), lambda i: (i, 0))
    wspec = pl.BlockSpec((D, D), lambda i: (0, 0))
    bspec = pl.BlockSpec((1, D), lambda i: (0, 0))
    return pl.pallas_call(
        _k4_body,
        grid=(nblk,),
        in_specs=[
            blkp, blkp, blkp,
            blkr, blkr,
            wspec, wspec, bspec,
            pl.BlockSpec((D, 2 * D), lambda i: (0, 0)),
            pl.BlockSpec((1, 2 * D), lambda i: (0, 0)),
            wspec, bspec,
        ],
        out_specs=pl.BlockSpec((1, 1, D), lambda i: (i, 0, 0)),
        out_shape=jax.ShapeDtypeStruct((nblk, 1, D), f32),
    )(x, qg, p, xr, pr, Wkg, Wv, bp1, Wp2cat, bcat, Wg2, bg2)


def _k5_body(pm_ref, wc, bc, o_ref):
    feat = jnp.max(pm_ref[...], axis=1)
    o_ref[...] = jnp.dot(feat, wc[...], preferred_element_type=jnp.float32) + bc[...]


def _classify(pmax, Wc, bc):
    return pl.pallas_call(
        _k5_body,
        out_shape=jax.ShapeDtypeStruct((B, NCLS), jnp.float32),
    )(pmax, Wc, bc.reshape(1, NCLS))


def kernel(features, xyz, W_embed, b_embed, Wq, Wk, Wv, Wp1, bp1, Wp2, bp2,
           Wg1, bg1, Wg2, bg2, Wc, bc):
    f32 = jnp.float32
    Wqg, Wkg, Wp2cat, bcat = _combine_weights(Wq, Wk, Wg1, Wp2, bp2, bg1)

    xyz_pad = jnp.pad(xyz, ((0, 0), (0, 0), (0, XP - 3)))
    xyzp_flat = xyz_pad.reshape(BN, XP)
    Wp1p = jnp.pad(Wp1, ((0, XP - 3), (0, 0)))

    f_flat = features.reshape(BN, C)
    x, qg, p = _project(f_flat, xyzp_flat, W_embed, b_embed, Wqg, Wp1p)

    HB = 1
    pmaxes = []
    for h in range(B // HB):
        gidx_h = _knn(xyz_pad[h * HB:(h + 1) * HB], base_b=h * HB)
        xr_h, pr_h = _sc_gather(x, p, gidx_h.reshape(HB * N * K))
        pmax_h = _pair_stage(x, qg, p, xr_h, pr_h, h * (HB * N // PB4),
                             Wkg, Wv, bp1.reshape(1, D), Wp2cat, bcat, Wg2,
                             bg2.reshape(1, D))
        pmaxes.append(pmax_h)
    pmax = jnp.concatenate(pmaxes, axis=0)
    logits = _classify(pmax.reshape(B, BN // PB4 // B, D), Wc, bc)
    return logits


# speedup vs baseline: 1.1523x; 1.1130x over previous
import functools

import jax
import jax.numpy as jnp
from jax import lax
from jax.experimental import pallas as pl
from jax.experimental.pallas import tpu as pltpu
from jax.experimental.pallas import tpu_sc as plsc

B, N, C, D, K, NCLS = 4, 2048, 128, 128, 16, 40
BN = B * N
BNK = B * N * K
XP = 16

def _k0_body(wq, wk, wg1, wp2, bp2, bg1, wqg, wkg, wp2cat, bcat):
    g1 = wg1[...]
    wqg[...] = jnp.dot(wq[...], g1, preferred_element_type=jnp.float32)
    wkg[...] = jnp.dot(wk[...], g1, preferred_element_type=jnp.float32)
    p2 = wp2[...]
    p2g = jnp.dot(p2, g1, preferred_element_type=jnp.float32)
    wp2cat[...] = jnp.concatenate([p2, p2g], axis=1)
    b2 = bp2[...]
    b2g = jnp.dot(b2, g1, preferred_element_type=jnp.float32) + bg1[...]
    bcat[...] = jnp.concatenate([b2, b2g], axis=1)


def _combine_weights(Wq, Wk, Wg1, Wp2, bp2, bg1):
    f32 = jnp.float32
    return pl.pallas_call(
        _k0_body,
        out_shape=(
            jax.ShapeDtypeStruct((D, D), f32),
            jax.ShapeDtypeStruct((D, D), f32),
            jax.ShapeDtypeStruct((D, 2 * D), f32),
            jax.ShapeDtypeStruct((1, 2 * D), f32),
        ),
    )(Wq, Wk, Wg1, Wp2, bp2.reshape(1, D), bg1.reshape(1, D))


PB1 = 2048

def _k1_body(f_ref, xyzp_ref, we, be, wqg, wp1, x_o, qg_o, p_o):
    x = jnp.dot(f_ref[...], we[...], preferred_element_type=jnp.float32) + be[...]
    x_o[...] = x
    qg_o[...] = jnp.dot(x, wqg[...], preferred_element_type=jnp.float32)
    p_o[...] = jnp.dot(xyzp_ref[...], wp1[...],
                       preferred_element_type=jnp.float32)


def _project(f_flat, xyzp_flat, W_embed, b_embed, Wqg, Wp1p):
    f32 = jnp.float32
    blk = pl.BlockSpec((PB1, D), lambda i: (i, 0))
    wspec = pl.BlockSpec((D, D), lambda i: (0, 0))
    return pl.pallas_call(
        _k1_body,
        grid=(BN // PB1,),
        in_specs=[blk, pl.BlockSpec((PB1, XP), lambda i: (i, 0)),
                  wspec, pl.BlockSpec((1, D), lambda i: (0, 0)),
                  wspec, pl.BlockSpec((XP, D), lambda i: (0, 0))],
        out_specs=(blk, blk, blk),
        out_shape=tuple(jax.ShapeDtypeStruct((BN, D), f32) for _ in range(3)),
    )(f_flat, xyzp_flat, W_embed, b_embed.reshape(1, D), Wqg, Wp1p)


PB2 = 256

def _k2_body(xyz_all_ref, xyzbt_ref, sq_ref, idx_o, *, base_b):
    b = pl.program_id(0) + base_b
    xa = xyz_all_ref[0]
    sq_blk = sq_ref[0]
    sq_all = jnp.sum(xa * xa, axis=1, keepdims=True)
    cross = jnp.dot(xa, xyzbt_ref[0], preferred_element_type=jnp.float32)
    d2 = jnp.maximum(sq_all + sq_blk - 2.0 * cross, 0.0)
    row = lax.broadcasted_iota(jnp.int32, (N, PB2), 0)
    key = (lax.bitcast_convert_type(d2, jnp.int32) & jnp.int32(~2047)) | row
    big = jnp.int32(2147483647)

    NS = N // 128
    slices = [key[s * 128:(s + 1) * 128, :] for s in range(NS)]
    m1 = slices[0]
    for s in range(1, NS):
        m1 = jnp.minimum(m1, slices[s])
    m2 = m3 = m4 = None
    prev = slices
    for lvl in range(3):
        cur_min = (m1, m2, m3)[lvl]
        nxt = [jnp.where(sl == cur_min, big, sl) for sl in prev]
        mm = nxt[0]
        for s in range(1, NS):
            mm = jnp.minimum(mm, nxt[s])
        if lvl == 0:
            m2 = mm
        elif lvl == 1:
            m3 = mm
        else:
            m4 = mm
        prev = nxt
    pool = jnp.concatenate([m1, m2, m3, m4], axis=0)
    ams = []
    for _ in range(K):
        m = jnp.min(pool, axis=0, keepdims=True)
        ams.append(m)
        pool = jnp.where(pool == m, big, pool)
    fast = jnp.concatenate(ams, axis=0)
    bad = jnp.any(m4 < ams[-1])

    def _slow(kk):
        outs = []
        for _ in range(K):
            mm = jnp.min(kk, axis=0, keepdims=True)
            outs.append(mm)
            kk = jnp.where(kk == mm, big, kk)
        return jnp.concatenate(outs, axis=0)

    keys16 = lax.cond(bad, lambda: _slow(key), lambda: fast)
    idx16 = (keys16 & jnp.int32(2047)) + b * N
    idx_o[0] = idx16.T


def _knn(xyz_pad, base_b):
    Bh = xyz_pad.shape[0]
    xyzt = jnp.swapaxes(xyz_pad, 1, 2)
    sq = jnp.sum(xyz_pad * xyz_pad, axis=2)[:, None, :]
    return pl.pallas_call(
        functools.partial(_k2_body, base_b=base_b),
        grid=(Bh, N // PB2),
        in_specs=[
            pl.BlockSpec((1, N, XP), lambda b, i: (b, 0, 0)),
            pl.BlockSpec((1, XP, PB2), lambda b, i: (b, 0, i)),
            pl.BlockSpec((1, 1, PB2), lambda b, i: (b, 0, i)),
        ],
        out_specs=pl.BlockSpec((1, PB2, K), lambda b, i: (b, i, 0)),
        out_shape=jax.ShapeDtypeStruct((Bh, N, K), jnp.int32),
    )(xyz_pad, xyzt, sq)


GW = 128

def _sc_gather(x, p, gidx):
    f32 = jnp.float32
    nidx = gidx.shape[0]
    mesh = plsc.VectorSubcoreMesh(core_axis_name="c", subcore_axis_name="s")
    idx2 = gidx.reshape(1, nidx)

    @functools.partial(
        pl.kernel,
        out_type=(
            jax.ShapeDtypeStruct((nidx, D), f32),
            jax.ShapeDtypeStruct((nidx, D), f32),
        ),
        mesh=mesh,
    )
    def gather_kernel(x_hbm, p_hbm, i_hbm, ox_hbm, op_hbm):
        def body(i_vmem, ox_vmem, op_vmem):
            pltpu.sync_copy(x_hbm.at[i_vmem.at[0]], ox_vmem)
            pltpu.sync_copy(p_hbm.at[i_vmem.at[0]], op_vmem)

        pltpu.emit_pipeline(
            body,
            grid=(nidx // GW,),
            in_specs=[pl.BlockSpec((1, GW), lambda i: (0, i))],
            out_specs=[
                pl.BlockSpec((GW, D), lambda i: (i, 0)),
                pl.BlockSpec((GW, D), lambda i: (i, 0)),
            ],
            core_axis_name=("c", "s"),
            dimension_semantics=(pltpu.PARALLEL,),
        )(i_hbm, ox_hbm, op_hbm)

    return gather_kernel(x, p, idx2)


PB4 = 128

def _k4_body(x_ref, qg_ref, pb_ref, xr_ref, pr_ref,
             wkg, wv, bp1, wp2cat, bcat, wg2, bg2, pm_ref):
    f32 = jnp.float32
    xr = xr_ref[...]
    kgr = jnp.dot(xr, wkg[...], preferred_element_type=f32)
    vr = jnp.dot(xr, wv[...], preferred_element_type=f32)
    p_i = pb_ref[...].reshape(PB4, 1, D) + bp1[...]
    h1 = jnp.maximum(
        (p_i - pr_ref[...].reshape(PB4, K, D)).reshape(PB4 * K, D), 0.0)
    hcat = jnp.dot(h1, wp2cat[...], preferred_element_type=f32) + bcat[...]
    pe = hcat[:, :D]
    pg = hcat[:, D:]
    qgb = jnp.broadcast_to(qg_ref[...].reshape(PB4, 1, D),
                           (PB4, K, D)).reshape(PB4 * K, D)
    a = jnp.maximum(qgb - kgr + pg, 0.0)
    u = jnp.dot(a, wg2[...], preferred_element_type=f32) + bg2[...]
    u3 = u.reshape(PB4, K, D)
    mx = jnp.max(u3, axis=1, keepdims=True)
    e = jnp.exp(u3 - mx)
    s = jnp.sum(e, axis=1)
    wsum = jnp.sum(e * (vr + pe).reshape(PB4, K, D), axis=1)
    out = wsum / s + x_ref[...]
    pm_ref[...] = jnp.max(out, axis=0, keepdims=True)[None]


def _pair_stage(x, qg, p, xr, pr, base_blk, Wkg, Wv, bp1, Wp2cat, bcat, Wg2,
                bg2):
    f32 = jnp.float32
    nblk = xr.shape[0] // (PB4 * K)
    blkp = pl.BlockSpec((PB4, D), lambda i: (base_blk + i, 0))
    blkr = pl.BlockSpec((PB4 * K, D), lambda i: (i, 0))
    wspec = pl.BlockSpec((D, D), lambda i: (0, 0))
    bspec = pl.BlockSpec((1, D), lambda i: (0, 0))
    return pl.pallas_call(
        _k4_body,
        grid=(nblk,),
        in_specs=[
            blkp, blkp, blkp,
            blkr, blkr,
            wspec, wspec, bspec,
            pl.BlockSpec((D, 2 * D), lambda i: (0, 0)),
            pl.BlockSpec((1, 2 * D), lambda i: (0, 0)),
            wspec, bspec,
        ],
        out_specs=pl.BlockSpec((1, 1, D), lambda i: (i, 0, 0)),
        out_shape=jax.ShapeDtypeStruct((nblk, 1, D), f32),
    )(x, qg, p, xr, pr, Wkg, Wv, bp1, Wp2cat, bcat, Wg2, bg2)


def _k5_body(pm_ref, wc, bc, o_ref):
    feat = jnp.max(pm_ref[...], axis=1)
    o_ref[...] = jnp.dot(feat, wc[...], preferred_element_type=jnp.float32) + bc[...]


def _classify(pmax, Wc, bc):
    return pl.pallas_call(
        _k5_body,
        out_shape=jax.ShapeDtypeStruct((B, NCLS), jnp.float32),
    )(pmax, Wc, bc.reshape(1, NCLS))


def kernel(features, xyz, W_embed, b_embed, Wq, Wk, Wv, Wp1, bp1, Wp2, bp2,
           Wg1, bg1, Wg2, bg2, Wc, bc):
    f32 = jnp.float32
    Wqg, Wkg, Wp2cat, bcat = _combine_weights(Wq, Wk, Wg1, Wp2, bp2, bg1)

    xyz_pad = jnp.pad(xyz, ((0, 0), (0, 0), (0, XP - 3)))
    xyzp_flat = xyz_pad.reshape(BN, XP)
    Wp1p = jnp.pad(Wp1, ((0, XP - 3), (0, 0)))

    f_flat = features.reshape(BN, C)
    x, qg, p = _project(f_flat, xyzp_flat, W_embed, b_embed, Wqg, Wp1p)

    HB = 1
    pmaxes = []
    for h in range(B // HB):
        gidx_h = _knn(xyz_pad[h * HB:(h + 1) * HB], base_b=h * HB)
        xr_h, pr_h = _sc_gather(x, p, gidx_h.reshape(HB * N * K))
        pmax_h = _pair_stage(x, qg, p, xr_h, pr_h, h * (HB * N // PB4),
                             Wkg, Wv, bp1.reshape(1, D), Wp2cat, bcat, Wg2,
                             bg2.reshape(1, D))
        pmaxes.append(pmax_h)
    pmax = jnp.concatenate(pmaxes, axis=0)
    logits = _classify(pmax.reshape(B, BN // PB4 // B, D), Wc, bc)
    return logits


# speedup vs baseline: 1.1531x; 1.0007x over previous
import functools

import jax
import jax.numpy as jnp
from jax import lax
from jax.experimental import pallas as pl
from jax.experimental.pallas import tpu as pltpu
from jax.experimental.pallas import tpu_sc as plsc

B, N, C, D, K, NCLS = 4, 2048, 128, 128, 16, 40
BN = B * N
BNK = B * N * K
XP = 16

def _k0_body(wq, wk, wg1, wp2, bp2, bg1, wqg, wkg, wp2cat, bcat):
    g1 = wg1[...]
    wqg[...] = jnp.dot(wq[...], g1, preferred_element_type=jnp.float32)
    wkg[...] = jnp.dot(wk[...], g1, preferred_element_type=jnp.float32)
    p2 = wp2[...]
    p2g = jnp.dot(p2, g1, preferred_element_type=jnp.float32)
    wp2cat[...] = jnp.concatenate([p2, p2g], axis=1)
    b2 = bp2[...]
    b2g = jnp.dot(b2, g1, preferred_element_type=jnp.float32) + bg1[...]
    bcat[...] = jnp.concatenate([b2, b2g], axis=1)


def _combine_weights(Wq, Wk, Wg1, Wp2, bp2, bg1):
    f32 = jnp.float32
    return pl.pallas_call(
        _k0_body,
        out_shape=(
            jax.ShapeDtypeStruct((D, D), f32),
            jax.ShapeDtypeStruct((D, D), f32),
            jax.ShapeDtypeStruct((D, 2 * D), f32),
            jax.ShapeDtypeStruct((1, 2 * D), f32),
        ),
    )(Wq, Wk, Wg1, Wp2, bp2.reshape(1, D), bg1.reshape(1, D))


PB1 = 2048

def _k1_body(f_ref, xyzp_ref, we, be, wqg, wp1, x_o, qg_o, p_o):
    x = jnp.dot(f_ref[...], we[...], preferred_element_type=jnp.float32) + be[...]
    x_o[...] = x
    qg_o[...] = jnp.dot(x, wqg[...], preferred_element_type=jnp.float32)
    p_o[...] = jnp.dot(xyzp_ref[...], wp1[...],
                       preferred_element_type=jnp.float32)


def _project(f_flat, xyzp_flat, W_embed, b_embed, Wqg, Wp1p):
    f32 = jnp.float32
    blk = pl.BlockSpec((PB1, D), lambda i: (i, 0))
    wspec = pl.BlockSpec((D, D), lambda i: (0, 0))
    return pl.pallas_call(
        _k1_body,
        grid=(BN // PB1,),
        in_specs=[blk, pl.BlockSpec((PB1, XP), lambda i: (i, 0)),
                  wspec, pl.BlockSpec((1, D), lambda i: (0, 0)),
                  wspec, pl.BlockSpec((XP, D), lambda i: (0, 0))],
        out_specs=(blk, blk, blk),
        out_shape=tuple(jax.ShapeDtypeStruct((BN, D), f32) for _ in range(3)),
    )(f_flat, xyzp_flat, W_embed, b_embed.reshape(1, D), Wqg, Wp1p)


PB2 = 512

def _k2_body(xyz_all_ref, xyzbt_ref, sq_ref, idx_o, *, base_b):
    b = pl.program_id(0) + base_b
    xa = xyz_all_ref[0]
    sq_blk = sq_ref[0]
    sq_all = jnp.sum(xa * xa, axis=1, keepdims=True)
    cross = jnp.dot(xa, xyzbt_ref[0], preferred_element_type=jnp.float32)
    d2 = jnp.maximum(sq_all + sq_blk - 2.0 * cross, 0.0)
    row = lax.broadcasted_iota(jnp.int32, (N, PB2), 0)
    key = (lax.bitcast_convert_type(d2, jnp.int32) & jnp.int32(~2047)) | row
    big = jnp.int32(2147483647)

    NS = N // 128
    slices = [key[s * 128:(s + 1) * 128, :] for s in range(NS)]
    m1 = slices[0]
    for s in range(1, NS):
        m1 = jnp.minimum(m1, slices[s])
    m2 = m3 = m4 = None
    prev = slices
    for lvl in range(3):
        cur_min = (m1, m2, m3)[lvl]
        nxt = [jnp.where(sl == cur_min, big, sl) for sl in prev]
        mm = nxt[0]
        for s in range(1, NS):
            mm = jnp.minimum(mm, nxt[s])
        if lvl == 0:
            m2 = mm
        elif lvl == 1:
            m3 = mm
        else:
            m4 = mm
        prev = nxt
    pool = jnp.concatenate([m1, m2, m3, m4], axis=0)
    ams = []
    for _ in range(K):
        m = jnp.min(pool, axis=0, keepdims=True)
        ams.append(m)
        pool = jnp.where(pool == m, big, pool)
    fast = jnp.concatenate(ams, axis=0)
    bad = jnp.any(m4 < ams[-1])

    def _slow(kk):
        outs = []
        for _ in range(K):
            mm = jnp.min(kk, axis=0, keepdims=True)
            outs.append(mm)
            kk = jnp.where(kk == mm, big, kk)
        return jnp.concatenate(outs, axis=0)

    keys16 = lax.cond(bad, lambda: _slow(key), lambda: fast)
    idx16 = (keys16 & jnp.int32(2047)) + b * N
    idx_o[0] = idx16.T


def _knn(xyz_pad, base_b):
    Bh = xyz_pad.shape[0]
    xyzt = jnp.swapaxes(xyz_pad, 1, 2)
    sq = jnp.sum(xyz_pad * xyz_pad, axis=2)[:, None, :]
    return pl.pallas_call(
        functools.partial(_k2_body, base_b=base_b),
        grid=(Bh, N // PB2),
        in_specs=[
            pl.BlockSpec((1, N, XP), lambda b, i: (b, 0, 0)),
            pl.BlockSpec((1, XP, PB2), lambda b, i: (b, 0, i)),
            pl.BlockSpec((1, 1, PB2), lambda b, i: (b, 0, i)),
        ],
        out_specs=pl.BlockSpec((1, PB2, K), lambda b, i: (b, i, 0)),
        out_shape=jax.ShapeDtypeStruct((Bh, N, K), jnp.int32),
    )(xyz_pad, xyzt, sq)


GW = 128

def _sc_gather(x, p, gidx):
    f32 = jnp.float32
    nidx = gidx.shape[0]
    mesh = plsc.VectorSubcoreMesh(core_axis_name="c", subcore_axis_name="s")
    idx2 = gidx.reshape(1, nidx)

    @functools.partial(
        pl.kernel,
        out_type=(
            jax.ShapeDtypeStruct((nidx, D), f32),
            jax.ShapeDtypeStruct((nidx, D), f32),
        ),
        mesh=mesh,
    )
    def gather_kernel(x_hbm, p_hbm, i_hbm, ox_hbm, op_hbm):
        def body(i_vmem, ox_vmem, op_vmem):
            pltpu.sync_copy(x_hbm.at[i_vmem.at[0]], ox_vmem)
            pltpu.sync_copy(p_hbm.at[i_vmem.at[0]], op_vmem)

        pltpu.emit_pipeline(
            body,
            grid=(nidx // GW,),
            in_specs=[pl.BlockSpec((1, GW), lambda i: (0, i))],
            out_specs=[
                pl.BlockSpec((GW, D), lambda i: (i, 0)),
                pl.BlockSpec((GW, D), lambda i: (i, 0)),
            ],
            core_axis_name=("c", "s"),
            dimension_semantics=(pltpu.PARALLEL,),
        )(i_hbm, ox_hbm, op_hbm)

    return gather_kernel(x, p, idx2)


PB4 = 256

def _k4_body(x_ref, qg_ref, pb_ref, xr_ref, pr_ref,
             wkg, wv, bp1, wp2cat, bcat, wg2, bg2, pm_ref):
    f32 = jnp.float32
    xr = xr_ref[...]
    kgr = jnp.dot(xr, wkg[...], preferred_element_type=f32)
    vr = jnp.dot(xr, wv[...], preferred_element_type=f32)
    p_i = pb_ref[...].reshape(PB4, 1, D) + bp1[...]
    h1 = jnp.maximum(
        (p_i - pr_ref[...].reshape(PB4, K, D)).reshape(PB4 * K, D), 0.0)
    hcat = jnp.dot(h1, wp2cat[...], preferred_element_type=f32) + bcat[...]
    pe = hcat[:, :D]
    pg = hcat[:, D:]
    qgb = jnp.broadcast_to(qg_ref[...].reshape(PB4, 1, D),
                           (PB4, K, D)).reshape(PB4 * K, D)
    a = jnp.maximum(qgb - kgr + pg, 0.0)
    u = jnp.dot(a, wg2[...], preferred_element_type=f32) + bg2[...]
    u3 = u.reshape(PB4, K, D)
    mx = jnp.max(u3, axis=1, keepdims=True)
    e = jnp.exp(u3 - mx)
    s = jnp.sum(e, axis=1)
    wsum = jnp.sum(e * (vr + pe).reshape(PB4, K, D), axis=1)
    out = wsum / s + x_ref[...]
    pm_ref[...] = jnp.max(out, axis=0, keepdims=True)[None]


def _pair_stage(x, qg, p, xr, pr, base_blk, Wkg, Wv, bp1, Wp2cat, bcat, Wg2,
                bg2):
    f32 = jnp.float32
    nblk = xr.shape[0] // (PB4 * K)
    blkp = pl.BlockSpec((PB4, D), lambda i: (base_blk + i, 0))
    blkr = pl.BlockSpec((PB4 * K, D), lambda i: (i, 0))
    wspec = pl.BlockSpec((D, D), lambda i: (0, 0))
    bspec = pl.BlockSpec((1, D), lambda i: (0, 0))
    return pl.pallas_call(
        _k4_body,
        grid=(nblk,),
        in_specs=[
            blkp, blkp, blkp,
            blkr, blkr,
            wspec, wspec, bspec,
            pl.BlockSpec((D, 2 * D), lambda i: (0, 0)),
            pl.BlockSpec((1, 2 * D), lambda i: (0, 0)),
            wspec, bspec,
        ],
        out_specs=pl.BlockSpec((1, 1, D), lambda i: (i, 0, 0)),
        out_shape=jax.ShapeDtypeStruct((nblk, 1, D), f32),
    )(x, qg, p, xr, pr, Wkg, Wv, bp1, Wp2cat, bcat, Wg2, bg2)


def _k5_body(pm_ref, wc, bc, o_ref):
    feat = jnp.max(pm_ref[...], axis=1)
    o_ref[...] = jnp.dot(feat, wc[...], preferred_element_type=jnp.float32) + bc[...]


def _classify(pmax, Wc, bc):
    return pl.pallas_call(
        _k5_body,
        out_shape=jax.ShapeDtypeStruct((B, NCLS), jnp.float32),
    )(pmax, Wc, bc.reshape(1, NCLS))


def kernel(features, xyz, W_embed, b_embed, Wq, Wk, Wv, Wp1, bp1, Wp2, bp2,
           Wg1, bg1, Wg2, bg2, Wc, bc):
    f32 = jnp.float32
    Wqg, Wkg, Wp2cat, bcat = _combine_weights(Wq, Wk, Wg1, Wp2, bp2, bg1)

    xyz_pad = jnp.pad(xyz, ((0, 0), (0, 0), (0, XP - 3)))
    xyzp_flat = xyz_pad.reshape(BN, XP)
    Wp1p = jnp.pad(Wp1, ((0, XP - 3), (0, 0)))

    f_flat = features.reshape(BN, C)
    x, qg, p = _project(f_flat, xyzp_flat, W_embed, b_embed, Wqg, Wp1p)

    HB = 1
    pmaxes = []
    for h in range(B // HB):
        gidx_h = _knn(xyz_pad[h * HB:(h + 1) * HB], base_b=h * HB)
        xr_h, pr_h = _sc_gather(x, p, gidx_h.reshape(HB * N * K))
        pmax_h = _pair_stage(x, qg, p, xr_h, pr_h, h * (HB * N // PB4),
                             Wkg, Wv, bp1.reshape(1, D), Wp2cat, bcat, Wg2,
                             bg2.reshape(1, D))
        pmaxes.append(pmax_h)
    pmax = jnp.concatenate(pmaxes, axis=0)
    logits = _classify(pmax.reshape(B, BN // PB4 // B, D), Wc, bc)
    return logits
